# Initial kernel scaffold; baseline (speedup 1.0000x reference)
import jax
import jax.numpy as jnp
from jax.experimental import pallas as pl


def kernel(node_feats, edge_index, edge_attr, W1, b1, W2, b2):
    raise NotImplementedError("write your pallas kernel here")


# speedup vs baseline: 2.6676x; 2.6676x over previous
import jax
import jax.numpy as jnp
from jax import lax
from jax.experimental import pallas as pl
from jax.experimental.pallas import tpu as pltpu
from jax.experimental.pallas import tpu_sc as plsc

N = 10000
D = 128
H = 64
E = 320000
L = 16
NS = 16
FPT = H // NS
C1 = 2560
NCH1 = E // C1
SUB1 = C1 // NS
EP = E // NS
C2 = 2000
NCH2 = EP // C2
NPAD = 10240
TSUB = NPAD // NS
NEG = -3.0e38


def _proj_body(x_ref, w1_ref, b1_ref, at_ref, bt_ref):
    x = x_ref[...]
    w1 = w1_ref[...]
    wa = w1[:D, :]
    wb = w1[D:, :]
    dn = (((0,), (1,)), ((), ()))
    at = lax.dot_general(wa, x, dn, preferred_element_type=jnp.float32)
    bcol = jnp.transpose(b1_ref[...])
    at_ref[...] = at + bcol
    bt_ref[...] = lax.dot_general(wb, x, dn, preferred_element_type=jnp.float32)


def _agg_body(t_ref, x_ref, o_ref):
    t = t_ref[...]
    x = x_ref[...]
    dn = (((1,), (0,)), ((), ()))
    o_ref[...] = lax.dot_general(t, x, dn, preferred_element_type=jnp.float32) * (1.0 / N)


def _sc_body(at_hbm, bt_hbm, src_hbm, dst_hbm, ea_hbm, w2_hbm, b2_hbm,
             w_out, t_out,
             atab, btab, srcb, dstb, accb, rb16, sbuf, eabuf, dst2b,
             tloc, redb, w2b, b2b, tmp16,
             part_sh, red_sh):
    s = lax.axis_index("s")

    pltpu.sync_copy(at_hbm.at[pl.ds(FPT * N * s, FPT * N)], atab)
    pltpu.sync_copy(bt_hbm.at[pl.ds(FPT * N * s, FPT * N)], btab)
    pltpu.sync_copy(w2_hbm, w2b)
    pltpu.sync_copy(b2_hbm, b2b)
    w2v = [plsc.load_gather(w2b, [jnp.full((L,), FPT * s + j, jnp.int32)])
           for j in range(FPT)]
    joff = [jnp.full((L,), j * N, jnp.int32) for j in range(FPT)]

    def chunk_body(ch, _):
        base = ch * C1
        pltpu.sync_copy(src_hbm.at[pl.ds(base, C1)], srcb)
        pltpu.sync_copy(dst_hbm.at[pl.ds(base, C1)], dstb)

        def grp(g, __):
            sv = srcb[pl.ds(g * L, L)]
            dv = dstb[pl.ds(g * L, L)]
            acc = jnp.zeros((L,), jnp.float32)
            for j in range(FPT):
                ga = plsc.load_gather(atab, [sv + joff[j]])
                gb = plsc.load_gather(btab, [dv + joff[j]])
                h = ga + gb
                lk = jnp.where(h >= 0, h, h * 0.2)
                acc = acc + w2v[j] * lk
            accb[pl.ds(g * L, L)] = acc
            return 0

        lax.fori_loop(0, C1 // L, grp, 0)
        pltpu.sync_copy(accb, part_sh.at[pl.ds(s * C1, C1)])
        plsc.subcore_barrier()
        for r in range(NS):
            pltpu.sync_copy(part_sh.at[pl.ds(r * C1 + s * SUB1, SUB1)],
                            rb16.at[pl.ds(r * SUB1, SUB1)])

        def colsum(c, __):
            v = rb16[pl.ds(c * L, L)]
            for r in range(1, NS):
                v = v + rb16[pl.ds(r * SUB1 + c * L, L)]
            sbuf[pl.ds(c * L, L)] = v
            return 0

        lax.fori_loop(0, SUB1 // L, colsum, 0)
        pltpu.sync_copy(sbuf.at[pl.ds(0, SUB1)],
                        w_out.at[pl.ds(base + s * SUB1, SUB1)])
        plsc.subcore_barrier()
        return 0

    lax.fori_loop(0, NCH1, chunk_body, 0)

    b2vec = b2b[...]
    base2 = s * EP

    def mod_chunk(k, mvec):
        sb = base2 + k * C2
        pltpu.sync_copy(w_out.at[pl.ds(sb, C2)], sbuf)
        pltpu.sync_copy(ea_hbm.at[pl.ds(sb, C2)], eabuf)

        def grp(g, mv):
            sc = eabuf[pl.ds(g * L, L)] * (sbuf[pl.ds(g * L, L)] + b2vec)
            sbuf[pl.ds(g * L, L)] = sc
            return jnp.maximum(mv, sc)

        mvec = lax.fori_loop(0, C2 // L, grp, mvec)
        pltpu.sync_copy(sbuf, w_out.at[pl.ds(sb, C2)])
        return mvec

    mvec = lax.fori_loop(0, NCH2, mod_chunk,
                         jnp.full((L,), NEG, jnp.float32))
    tmp16[...] = jnp.full((L,), jnp.max(mvec), jnp.float32)
    pltpu.sync_copy(tmp16, red_sh.at[pl.ds(s * L, L)])
    plsc.subcore_barrier()
    pltpu.sync_copy(red_sh.at[pl.ds(0, NS * L)], redb)
    mv = redb[pl.ds(0, L)]
    for r in range(1, NS):
        mv = jnp.maximum(mv, redb[pl.ds(r * L, L)])
    gmvec = mv

    def exp_chunk(k, zvec):
        sb = base2 + k * C2
        pltpu.sync_copy(w_out.at[pl.ds(sb, C2)], sbuf)

        def grp(g, zv):
            ex = jnp.exp(sbuf[pl.ds(g * L, L)] - gmvec)
            sbuf[pl.ds(g * L, L)] = ex
            return zv + ex

        zvec = lax.fori_loop(0, C2 // L, grp, zvec)
        pltpu.sync_copy(sbuf, w_out.at[pl.ds(sb, C2)])
        return zvec

    zvec = lax.fori_loop(0, NCH2, exp_chunk, jnp.zeros((L,), jnp.float32))
    tmp16[...] = jnp.full((L,), jnp.sum(zvec), jnp.float32)
    pltpu.sync_copy(tmp16, red_sh.at[pl.ds(NS * L + s * L, L)])
    plsc.subcore_barrier()
    pltpu.sync_copy(red_sh.at[pl.ds(NS * L, NS * L)], redb)
    zv = redb[pl.ds(0, L)]
    for r in range(1, NS):
        zv = zv + redb[pl.ds(r * L, L)]
    invvec = jnp.ones((L,), jnp.float32) / zv

    def zt(i, _):
        tloc[pl.ds(i * L, L)] = jnp.zeros((L,), jnp.float32)
        return 0

    lax.fori_loop(0, NPAD // L, zt, 0)

    def norm_chunk(k, _):
        sb = base2 + k * C2
        pltpu.sync_copy(w_out.at[pl.ds(sb, C2)], sbuf)
        pltpu.sync_copy(dst_hbm.at[pl.ds(sb, C2)], dst2b)

        def grp(g, __):
            wv = sbuf[pl.ds(g * L, L)] * invvec
            sbuf[pl.ds(g * L, L)] = wv
            dv = dst2b[pl.ds(g * L, L)]
            plsc.addupdate_scatter(tloc, [dv], wv)
            return 0

        lax.fori_loop(0, C2 // L, grp, 0)
        pltpu.sync_copy(sbuf, w_out.at[pl.ds(sb, C2)])
        return 0

    lax.fori_loop(0, NCH2, norm_chunk, 0)
    for r4 in range(NPAD // C1):
        pltpu.sync_copy(tloc.at[pl.ds(r4 * C1, C1)],
                        part_sh.at[pl.ds(s * C1, C1)])
        plsc.subcore_barrier()
        for r in range(NS):
            pltpu.sync_copy(part_sh.at[pl.ds(r * C1 + s * SUB1, SUB1)],
                            rb16.at[pl.ds(r * SUB1, SUB1)])

        def tsum(c, _):
            v = rb16[pl.ds(c * L, L)]
            for r in range(1, NS):
                v = v + rb16[pl.ds(r * SUB1 + c * L, L)]
            sbuf[pl.ds(c * L, L)] = v
            return 0

        lax.fori_loop(0, SUB1 // L, tsum, 0)
        pltpu.sync_copy(sbuf.at[pl.ds(0, SUB1)],
                        t_out.at[pl.ds(r4 * C1 + s * SUB1, SUB1)])
        plsc.subcore_barrier()


@jax.jit
def kernel(node_feats, edge_index, edge_attr, W1, b1, W2, b2):
    x = node_feats[0]
    src = edge_index[0]
    dst = edge_index[1]
    ea = edge_attr[:, 0]
    w2f = W2[:, 0]
    b1r = b1.reshape(1, H)
    b2v = jnp.broadcast_to(b2.astype(jnp.float32), (L,))

    at, bt = pl.pallas_call(
        _proj_body,
        out_shape=(
            jax.ShapeDtypeStruct((H, N), jnp.float32),
            jax.ShapeDtypeStruct((H, N), jnp.float32),
        ),
    )(x, W1, b1r)

    mesh = plsc.VectorSubcoreMesh(core_axis_name="c", subcore_axis_name="s",
                                  num_cores=1, num_subcores=NS)
    w, t = pl.kernel(
        _sc_body,
        out_type=(
            jax.ShapeDtypeStruct((E,), jnp.float32),
            jax.ShapeDtypeStruct((NPAD,), jnp.float32),
        ),
        mesh=mesh,
        compiler_params=pltpu.CompilerParams(needs_layout_passes=False),
        scratch_types=[
            pltpu.VMEM((FPT * N,), jnp.float32),
            pltpu.VMEM((FPT * N,), jnp.float32),
            pltpu.VMEM((C1,), jnp.int32),
            pltpu.VMEM((C1,), jnp.int32),
            pltpu.VMEM((C1,), jnp.float32),
            pltpu.VMEM((C1,), jnp.float32),
            pltpu.VMEM((C2,), jnp.float32),
            pltpu.VMEM((C2,), jnp.float32),
            pltpu.VMEM((C2,), jnp.int32),
            pltpu.VMEM((NPAD,), jnp.float32),
            pltpu.VMEM((NS * L,), jnp.float32),
            pltpu.VMEM((H,), jnp.float32),
            pltpu.VMEM((L,), jnp.float32),
            pltpu.VMEM((L,), jnp.float32),
            pltpu.VMEM_SHARED((NS * C1,), jnp.float32),
            pltpu.VMEM_SHARED((2 * NS * L,), jnp.float32),
        ],
    )(at.reshape(H * N), bt.reshape(H * N), src, dst, ea, w2f, b2v)

    gf = pl.pallas_call(
        _agg_body,
        out_shape=jax.ShapeDtypeStruct((1, D), jnp.float32),
    )(t[:N].reshape(1, N), x)

    return gf, w.reshape(1, E)


# speedup vs baseline: 4.0830x; 1.5306x over previous
import jax
import jax.numpy as jnp
from jax import lax
from jax.experimental import pallas as pl
from jax.experimental.pallas import tpu as pltpu
from jax.experimental.pallas import tpu_sc as plsc

N = 10000
D = 128
H = 64
E = 320000
L = 16
NS = 16
FPT = H // NS
C1 = 6400
NCH1 = E // C1
SUB1 = C1 // NS
TC1 = 2560
TSUB = TC1 // NS
EP = E // NS
C2 = 2000
NCH2 = EP // C2
NPAD = 10240
NEG = -3.0e38


def _proj_body(x_ref, w1_ref, b1_ref, at_ref, bt_ref):
    x = x_ref[...]
    w1 = w1_ref[...]
    wa = w1[:D, :]
    wb = w1[D:, :]
    dn = (((0,), (1,)), ((), ()))
    at = lax.dot_general(wa, x, dn, preferred_element_type=jnp.float32)
    bcol = jnp.transpose(b1_ref[...])
    at_ref[...] = at + bcol
    bt_ref[...] = lax.dot_general(wb, x, dn, preferred_element_type=jnp.float32)


def _agg_body(t_ref, x_ref, o_ref):
    t = t_ref[...]
    x = x_ref[...]
    dn = (((1,), (0,)), ((), ()))
    o_ref[...] = lax.dot_general(t, x, dn, preferred_element_type=jnp.float32) * (1.0 / N)


def _sc_body(at_hbm, bt_hbm, src_hbm, dst_hbm, ea_hbm, w2_hbm, b2_hbm,
             w_out, t_out,
             atab, btab, srcb, dstb, accb, rb16, sbuf, eabuf, dst2b,
             tloc, redb, w2b, b2b, tmp16, dsem,
             part_sh, red_sh):
    s = lax.axis_index("s")

    pltpu.sync_copy(at_hbm.at[pl.ds(FPT * N * s, FPT * N)], atab)
    pltpu.sync_copy(bt_hbm.at[pl.ds(FPT * N * s, FPT * N)], btab)
    pltpu.sync_copy(w2_hbm, w2b)
    pltpu.sync_copy(b2_hbm, b2b)
    w2v = [plsc.load_gather(w2b, [jnp.full((L,), FPT * s + j, jnp.int32)])
           for j in range(FPT)]
    joff = [jnp.full((L,), j * N, jnp.int32) for j in range(FPT)]

    def chunk_body(ch, _):
        base = ch * C1
        pltpu.sync_copy(src_hbm.at[pl.ds(base, C1)], srcb)
        pltpu.sync_copy(dst_hbm.at[pl.ds(base, C1)], dstb)

        def grp(g, __):
            sv = srcb[pl.ds(g * L, L)]
            dv = dstb[pl.ds(g * L, L)]
            acc = jnp.zeros((L,), jnp.float32)
            for j in range(FPT):
                ga = plsc.load_gather(atab, [sv + joff[j]])
                gb = plsc.load_gather(btab, [dv + joff[j]])
                h = ga + gb
                lk = jnp.where(h >= 0, h, h * 0.2)
                acc = acc + w2v[j] * lk
            accb[pl.ds(g * L, L)] = acc
            return 0

        lax.fori_loop(0, C1 // L, grp, 0)
        pltpu.sync_copy(accb, part_sh.at[pl.ds(s * C1, C1)])
        plsc.subcore_barrier()
        descs = [pltpu.async_copy(part_sh.at[pl.ds(r * C1 + s * SUB1, SUB1)],
                                  rb16.at[pl.ds(r * SUB1, SUB1)], dsem)
                 for r in range(NS)]
        for dsc in descs:
            dsc.wait()

        def colsum(c, __):
            v = rb16[pl.ds(c * L, L)]
            for r in range(1, NS):
                v = v + rb16[pl.ds(r * SUB1 + c * L, L)]
            sbuf[pl.ds(c * L, L)] = v
            return 0

        lax.fori_loop(0, SUB1 // L, colsum, 0)
        pltpu.sync_copy(sbuf.at[pl.ds(0, SUB1)],
                        w_out.at[pl.ds(base + s * SUB1, SUB1)])
        plsc.subcore_barrier()
        return 0

    lax.fori_loop(0, NCH1, chunk_body, 0)

    b2vec = b2b[...]
    base2 = s * EP

    def mod_chunk(k, mvec):
        sb = base2 + k * C2
        pltpu.sync_copy(w_out.at[pl.ds(sb, C2)], sbuf)
        pltpu.sync_copy(ea_hbm.at[pl.ds(sb, C2)], eabuf)

        def grp(g, mv):
            sc = eabuf[pl.ds(g * L, L)] * (sbuf[pl.ds(g * L, L)] + b2vec)
            sbuf[pl.ds(g * L, L)] = sc
            return jnp.maximum(mv, sc)

        mvec = lax.fori_loop(0, C2 // L, grp, mvec)
        pltpu.sync_copy(sbuf, w_out.at[pl.ds(sb, C2)])
        return mvec

    mvec = lax.fori_loop(0, NCH2, mod_chunk,
                         jnp.full((L,), NEG, jnp.float32))
    tmp16[...] = jnp.full((L,), jnp.max(mvec), jnp.float32)
    pltpu.sync_copy(tmp16, red_sh.at[pl.ds(s * L, L)])
    plsc.subcore_barrier()
    pltpu.sync_copy(red_sh.at[pl.ds(0, NS * L)], redb)
    mv = redb[pl.ds(0, L)]
    for r in range(1, NS):
        mv = jnp.maximum(mv, redb[pl.ds(r * L, L)])
    gmvec = mv

    def exp_chunk(k, zvec):
        sb = base2 + k * C2
        pltpu.sync_copy(w_out.at[pl.ds(sb, C2)], sbuf)

        def grp(g, zv):
            ex = jnp.exp(sbuf[pl.ds(g * L, L)] - gmvec)
            sbuf[pl.ds(g * L, L)] = ex
            return zv + ex

        zvec = lax.fori_loop(0, C2 // L, grp, zvec)
        pltpu.sync_copy(sbuf, w_out.at[pl.ds(sb, C2)])
        return zvec

    zvec = lax.fori_loop(0, NCH2, exp_chunk, jnp.zeros((L,), jnp.float32))
    tmp16[...] = jnp.full((L,), jnp.sum(zvec), jnp.float32)
    pltpu.sync_copy(tmp16, red_sh.at[pl.ds(NS * L + s * L, L)])
    plsc.subcore_barrier()
    pltpu.sync_copy(red_sh.at[pl.ds(NS * L, NS * L)], redb)
    zv = redb[pl.ds(0, L)]
    for r in range(1, NS):
        zv = zv + redb[pl.ds(r * L, L)]
    invvec = jnp.ones((L,), jnp.float32) / zv

    def zt(i, _):
        tloc[pl.ds(i * L, L)] = jnp.zeros((L,), jnp.float32)
        return 0

    lax.fori_loop(0, NPAD // L, zt, 0)

    def norm_chunk(k, _):
        sb = base2 + k * C2
        pltpu.sync_copy(w_out.at[pl.ds(sb, C2)], sbuf)
        pltpu.sync_copy(dst_hbm.at[pl.ds(sb, C2)], dst2b)

        def grp(g, __):
            wv = sbuf[pl.ds(g * L, L)] * invvec
            sbuf[pl.ds(g * L, L)] = wv
            dv = dst2b[pl.ds(g * L, L)]
            plsc.addupdate_scatter(tloc, [dv], wv)
            return 0

        lax.fori_loop(0, C2 // L, grp, 0)
        pltpu.sync_copy(sbuf, w_out.at[pl.ds(sb, C2)])
        return 0

    lax.fori_loop(0, NCH2, norm_chunk, 0)
    for r4 in range(NPAD // TC1):
        pltpu.sync_copy(tloc.at[pl.ds(r4 * TC1, TC1)],
                        part_sh.at[pl.ds(s * C1, TC1)])
        plsc.subcore_barrier()
        descs = [pltpu.async_copy(part_sh.at[pl.ds(r * C1 + s * TSUB, TSUB)],
                                  rb16.at[pl.ds(r * TSUB, TSUB)], dsem)
                 for r in range(NS)]
        for dsc in descs:
            dsc.wait()

        def tsum(c, _):
            v = rb16[pl.ds(c * L, L)]
            for r in range(1, NS):
                v = v + rb16[pl.ds(r * TSUB + c * L, L)]
            sbuf[pl.ds(c * L, L)] = v
            return 0

        lax.fori_loop(0, TSUB // L, tsum, 0)
        pltpu.sync_copy(sbuf.at[pl.ds(0, TSUB)],
                        t_out.at[pl.ds(r4 * TC1 + s * TSUB, TSUB)])
        plsc.subcore_barrier()


@jax.jit
def kernel(node_feats, edge_index, edge_attr, W1, b1, W2, b2):
    x = node_feats[0]
    src = edge_index[0]
    dst = edge_index[1]
    ea = edge_attr[:, 0]
    w2f = W2[:, 0]
    b1r = b1.reshape(1, H)
    b2v = jnp.broadcast_to(b2.astype(jnp.float32), (L,))

    at, bt = pl.pallas_call(
        _proj_body,
        out_shape=(
            jax.ShapeDtypeStruct((H, N), jnp.float32),
            jax.ShapeDtypeStruct((H, N), jnp.float32),
        ),
    )(x, W1, b1r)

    mesh = plsc.VectorSubcoreMesh(core_axis_name="c", subcore_axis_name="s",
                                  num_cores=1, num_subcores=NS)
    w, t = pl.kernel(
        _sc_body,
        out_type=(
            jax.ShapeDtypeStruct((E,), jnp.float32),
            jax.ShapeDtypeStruct((NPAD,), jnp.float32),
        ),
        mesh=mesh,
        compiler_params=pltpu.CompilerParams(needs_layout_passes=False),
        scratch_types=[
            pltpu.VMEM((FPT * N,), jnp.float32),
            pltpu.VMEM((FPT * N,), jnp.float32),
            pltpu.VMEM((C1,), jnp.int32),
            pltpu.VMEM((C1,), jnp.int32),
            pltpu.VMEM((C1,), jnp.float32),
            pltpu.VMEM((C1,), jnp.float32),
            pltpu.VMEM((C2,), jnp.float32),
            pltpu.VMEM((C2,), jnp.float32),
            pltpu.VMEM((C2,), jnp.int32),
            pltpu.VMEM((NPAD,), jnp.float32),
            pltpu.VMEM((NS * L,), jnp.float32),
            pltpu.VMEM((H,), jnp.float32),
            pltpu.VMEM((L,), jnp.float32),
            pltpu.VMEM((L,), jnp.float32),
            pltpu.SemaphoreType.DMA,
            pltpu.VMEM_SHARED((NS * C1,), jnp.float32),
            pltpu.VMEM_SHARED((2 * NS * L,), jnp.float32),
        ],
    )(at.reshape(H * N), bt.reshape(H * N), src, dst, ea, w2f, b2v)

    gf = pl.pallas_call(
        _agg_body,
        out_shape=jax.ShapeDtypeStruct((1, D), jnp.float32),
    )(t[:N].reshape(1, N), x)

    return gf, w.reshape(1, E)


# speedup vs baseline: 5.7115x; 1.3988x over previous
import jax
import jax.numpy as jnp
from jax import lax
from jax.experimental import pallas as pl
from jax.experimental.pallas import tpu as pltpu
from jax.experimental.pallas import tpu_sc as plsc

N = 10000
D = 128
H = 64
E = 320000
L = 16
NS = 16
FPT = H // NS
C1 = 6400
NCH1 = E // C1
SUB1 = C1 // NS
TC1 = 2560
TSUB = TC1 // NS
EP = E // NS
C2 = 2000
NCH2 = EP // C2
NPAD = 10240
NEG = -3.0e38


def _proj_body(x_ref, w1_ref, b1_ref, at_ref, bt_ref):
    x = x_ref[...]
    w1 = w1_ref[...]
    wa = w1[:D, :]
    wb = w1[D:, :]
    dn = (((0,), (1,)), ((), ()))
    at = lax.dot_general(wa, x, dn, preferred_element_type=jnp.float32)
    bcol = jnp.transpose(b1_ref[...])
    at_ref[...] = at + bcol
    bt_ref[...] = lax.dot_general(wb, x, dn, preferred_element_type=jnp.float32)


def _agg_body(t_ref, x_ref, o_ref):
    t = t_ref[...]
    x = x_ref[...]
    dn = (((1,), (0,)), ((), ()))
    o_ref[...] = lax.dot_general(t, x, dn, preferred_element_type=jnp.float32) * (1.0 / N)


def _sc_body(at_hbm, bt_hbm, src_hbm, dst_hbm, ea_hbm, w2_hbm, b2_hbm,
             w_out, t_out,
             atab, btab, srcb, dstb, accb, rb16, sbuf, eabuf, dst2b,
             tloc, redb, w2b, b2b, tmp16, dsem,
             part_sh, red_sh):
    s = lax.axis_index("s")

    pltpu.sync_copy(at_hbm.at[pl.ds(FPT * N * s, FPT * N)], atab)
    pltpu.sync_copy(bt_hbm.at[pl.ds(FPT * N * s, FPT * N)], btab)
    pltpu.sync_copy(w2_hbm, w2b)
    pltpu.sync_copy(b2_hbm, b2b)
    w2v = [plsc.load_gather(w2b, [jnp.full((L,), FPT * s + j, jnp.int32)])
           for j in range(FPT)]
    joff = [jnp.full((L,), j * N, jnp.int32) for j in range(FPT)]

    def chunk_body(ch, _):
        base = ch * C1
        pltpu.sync_copy(src_hbm.at[pl.ds(base, C1)], srcb)
        pltpu.sync_copy(dst_hbm.at[pl.ds(base, C1)], dstb)

        @plsc.parallel_loop(0, C1 // L, unroll=4)
        def grp(g):
            sv = srcb[pl.ds(g * L, L)]
            dv = dstb[pl.ds(g * L, L)]
            acc = jnp.zeros((L,), jnp.float32)
            for j in range(FPT):
                ga = plsc.load_gather(atab, [sv + joff[j]])
                gb = plsc.load_gather(btab, [dv + joff[j]])
                h = ga + gb
                lk = jnp.where(h >= 0, h, h * 0.2)
                acc = acc + w2v[j] * lk
            accb[pl.ds(g * L, L)] = acc
        pltpu.sync_copy(accb, part_sh.at[pl.ds(s * C1, C1)])
        plsc.subcore_barrier()
        descs = [pltpu.async_copy(part_sh.at[pl.ds(r * C1 + s * SUB1, SUB1)],
                                  rb16.at[pl.ds(r * SUB1, SUB1)], dsem)
                 for r in range(NS)]
        for dsc in descs:
            dsc.wait()

        @plsc.parallel_loop(0, SUB1 // L, unroll=2)
        def colsum(c):
            v = rb16[pl.ds(c * L, L)]
            for r in range(1, NS):
                v = v + rb16[pl.ds(r * SUB1 + c * L, L)]
            sbuf[pl.ds(c * L, L)] = v
        pltpu.sync_copy(sbuf.at[pl.ds(0, SUB1)],
                        w_out.at[pl.ds(base + s * SUB1, SUB1)])
        plsc.subcore_barrier()
        return 0

    lax.fori_loop(0, NCH1, chunk_body, 0)

    b2vec = b2b[...]
    base2 = s * EP

    def mod_chunk(k, mvec):
        sb = base2 + k * C2
        pltpu.sync_copy(w_out.at[pl.ds(sb, C2)], sbuf)
        pltpu.sync_copy(ea_hbm.at[pl.ds(sb, C2)], eabuf)

        @plsc.parallel_loop(0, C2 // L, unroll=4, carry=mvec)
        def grp(g, mv):
            sc = eabuf[pl.ds(g * L, L)] * (sbuf[pl.ds(g * L, L)] + b2vec)
            sbuf[pl.ds(g * L, L)] = sc
            return jnp.maximum(mv, sc)

        mvec = grp
        pltpu.sync_copy(sbuf, w_out.at[pl.ds(sb, C2)])
        return mvec

    mvec = lax.fori_loop(0, NCH2, mod_chunk,
                         jnp.full((L,), NEG, jnp.float32))
    tmp16[...] = jnp.full((L,), jnp.max(mvec), jnp.float32)
    pltpu.sync_copy(tmp16, red_sh.at[pl.ds(s * L, L)])
    plsc.subcore_barrier()
    pltpu.sync_copy(red_sh.at[pl.ds(0, NS * L)], redb)
    mv = redb[pl.ds(0, L)]
    for r in range(1, NS):
        mv = jnp.maximum(mv, redb[pl.ds(r * L, L)])
    gmvec = mv

    def exp_chunk(k, zvec):
        sb = base2 + k * C2
        pltpu.sync_copy(w_out.at[pl.ds(sb, C2)], sbuf)

        @plsc.parallel_loop(0, C2 // L, unroll=4, carry=zvec)
        def grp(g, zv):
            ex = jnp.exp(sbuf[pl.ds(g * L, L)] - gmvec)
            sbuf[pl.ds(g * L, L)] = ex
            return zv + ex

        zvec = grp
        pltpu.sync_copy(sbuf, w_out.at[pl.ds(sb, C2)])
        return zvec

    zvec = lax.fori_loop(0, NCH2, exp_chunk, jnp.zeros((L,), jnp.float32))
    tmp16[...] = jnp.full((L,), jnp.sum(zvec), jnp.float32)
    pltpu.sync_copy(tmp16, red_sh.at[pl.ds(NS * L + s * L, L)])
    plsc.subcore_barrier()
    pltpu.sync_copy(red_sh.at[pl.ds(NS * L, NS * L)], redb)
    zv = redb[pl.ds(0, L)]
    for r in range(1, NS):
        zv = zv + redb[pl.ds(r * L, L)]
    invvec = jnp.ones((L,), jnp.float32) / zv

    @plsc.parallel_loop(0, NPAD // L, unroll=4)
    def zt(i):
        tloc[pl.ds(i * L, L)] = jnp.zeros((L,), jnp.float32)

    def norm_chunk(k, _):
        sb = base2 + k * C2
        pltpu.sync_copy(w_out.at[pl.ds(sb, C2)], sbuf)
        pltpu.sync_copy(dst_hbm.at[pl.ds(sb, C2)], dst2b)

        @plsc.parallel_loop(0, C2 // L, unroll=4)
        def grp(g):
            wv = sbuf[pl.ds(g * L, L)] * invvec
            sbuf[pl.ds(g * L, L)] = wv
            dv = dst2b[pl.ds(g * L, L)]
            plsc.addupdate_scatter(tloc, [dv], wv)
        pltpu.sync_copy(sbuf, w_out.at[pl.ds(sb, C2)])
        return 0

    lax.fori_loop(0, NCH2, norm_chunk, 0)
    for r4 in range(NPAD // TC1):
        pltpu.sync_copy(tloc.at[pl.ds(r4 * TC1, TC1)],
                        part_sh.at[pl.ds(s * C1, TC1)])
        plsc.subcore_barrier()
        descs = [pltpu.async_copy(part_sh.at[pl.ds(r * C1 + s * TSUB, TSUB)],
                                  rb16.at[pl.ds(r * TSUB, TSUB)], dsem)
                 for r in range(NS)]
        for dsc in descs:
            dsc.wait()

        @plsc.parallel_loop(0, TSUB // L, unroll=2)
        def tsum(c):
            v = rb16[pl.ds(c * L, L)]
            for r in range(1, NS):
                v = v + rb16[pl.ds(r * TSUB + c * L, L)]
            sbuf[pl.ds(c * L, L)] = v
        pltpu.sync_copy(sbuf.at[pl.ds(0, TSUB)],
                        t_out.at[pl.ds(r4 * TC1 + s * TSUB, TSUB)])
        plsc.subcore_barrier()


@jax.jit
def kernel(node_feats, edge_index, edge_attr, W1, b1, W2, b2):
    x = node_feats[0]
    src = edge_index[0]
    dst = edge_index[1]
    ea = edge_attr[:, 0]
    w2f = W2[:, 0]
    b1r = b1.reshape(1, H)
    b2v = jnp.broadcast_to(b2.astype(jnp.float32), (L,))

    at, bt = pl.pallas_call(
        _proj_body,
        out_shape=(
            jax.ShapeDtypeStruct((H, N), jnp.float32),
            jax.ShapeDtypeStruct((H, N), jnp.float32),
        ),
    )(x, W1, b1r)

    mesh = plsc.VectorSubcoreMesh(core_axis_name="c", subcore_axis_name="s",
                                  num_cores=1, num_subcores=NS)
    w, t = pl.kernel(
        _sc_body,
        out_type=(
            jax.ShapeDtypeStruct((E,), jnp.float32),
            jax.ShapeDtypeStruct((NPAD,), jnp.float32),
        ),
        mesh=mesh,
        compiler_params=pltpu.CompilerParams(needs_layout_passes=False),
        scratch_types=[
            pltpu.VMEM((FPT * N,), jnp.float32),
            pltpu.VMEM((FPT * N,), jnp.float32),
            pltpu.VMEM((C1,), jnp.int32),
            pltpu.VMEM((C1,), jnp.int32),
            pltpu.VMEM((C1,), jnp.float32),
            pltpu.VMEM((C1,), jnp.float32),
            pltpu.VMEM((C2,), jnp.float32),
            pltpu.VMEM((C2,), jnp.float32),
            pltpu.VMEM((C2,), jnp.int32),
            pltpu.VMEM((NPAD,), jnp.float32),
            pltpu.VMEM((NS * L,), jnp.float32),
            pltpu.VMEM((H,), jnp.float32),
            pltpu.VMEM((L,), jnp.float32),
            pltpu.VMEM((L,), jnp.float32),
            pltpu.SemaphoreType.DMA,
            pltpu.VMEM_SHARED((NS * C1,), jnp.float32),
            pltpu.VMEM_SHARED((2 * NS * L,), jnp.float32),
        ],
    )(at.reshape(H * N), bt.reshape(H * N), src, dst, ea, w2f, b2v)

    gf = pl.pallas_call(
        _agg_body,
        out_shape=jax.ShapeDtypeStruct((1, D), jnp.float32),
    )(t[:N].reshape(1, N), x)

    return gf, w.reshape(1, E)


# speedup vs baseline: 9.1791x; 1.6071x over previous
import jax
import jax.numpy as jnp
from jax import lax
from jax.experimental import pallas as pl
from jax.experimental.pallas import tpu as pltpu
from jax.experimental.pallas import tpu_sc as plsc

N = 10000
D = 128
H = 64
E = 320000
L = 16
NC = 2
NS = 16
NW = NC * NS
FPT = H // NS
HALF = E // NC
C1 = 6400
NCH1 = HALF // C1
SUB1 = C1 // NS
EP = E // NW
C2 = 2000
NCH2 = EP // C2
NPAD = 10240
TC1 = 2560
TSUB = TC1 // NS
NEG = -3.0e38


def _proj_body(x_ref, w1_ref, b1_ref, at_ref, bt_ref):
    x = x_ref[...]
    w1 = w1_ref[...]
    wa = w1[:D, :]
    wb = w1[D:, :]
    dn = (((0,), (1,)), ((), ()))
    at = lax.dot_general(wa, x, dn, preferred_element_type=jnp.float32)
    bcol = jnp.transpose(b1_ref[...])
    at_ref[...] = at + bcol
    bt_ref[...] = lax.dot_general(wb, x, dn, preferred_element_type=jnp.float32)


def _agg_body(t2_ref, x_ref, o_ref):
    t2 = t2_ref[...]
    x = x_ref[...]
    t = t2[0:1, :N] + t2[1:2, :N]
    dn = (((1,), (0,)), ((), ()))
    o_ref[...] = lax.dot_general(t, x, dn, preferred_element_type=jnp.float32) * (1.0 / N)


def _sc_a_body(at_hbm, bt_hbm, src_hbm, dst_hbm, ea_hbm, w2_hbm, b2_hbm,
               sc_out, ms_out,
               atab, btab, srcb, dstb, accb, rb16, sbuf, eabuf,
               w2b, b2b, tmp16, dsem,
               part_sh):
    c = lax.axis_index("c")
    s = lax.axis_index("s")
    wid = c * NS + s

    pltpu.sync_copy(at_hbm.at[pl.ds(FPT * N * s, FPT * N)], atab)
    pltpu.sync_copy(bt_hbm.at[pl.ds(FPT * N * s, FPT * N)], btab)
    pltpu.sync_copy(w2_hbm, w2b)
    pltpu.sync_copy(b2_hbm, b2b)
    w2v = [plsc.load_gather(w2b, [jnp.full((L,), FPT * s + j, jnp.int32)])
           for j in range(FPT)]
    joff = [jnp.full((L,), j * N, jnp.int32) for j in range(FPT)]

    def chunk_body(ch, _):
        base = c * HALF + ch * C1
        pltpu.sync_copy(src_hbm.at[pl.ds(base, C1)], srcb)
        pltpu.sync_copy(dst_hbm.at[pl.ds(base, C1)], dstb)

        @plsc.parallel_loop(0, C1 // L, unroll=4)
        def grp(g):
            sv = srcb[pl.ds(g * L, L)]
            dv = dstb[pl.ds(g * L, L)]
            acc = jnp.zeros((L,), jnp.float32)
            for j in range(FPT):
                ga = plsc.load_gather(atab, [sv + joff[j]])
                gb = plsc.load_gather(btab, [dv + joff[j]])
                h = ga + gb
                lk = jnp.where(h >= 0, h, h * 0.2)
                acc = acc + w2v[j] * lk
            accb[pl.ds(g * L, L)] = acc

        pltpu.sync_copy(accb, part_sh.at[pl.ds(s * C1, C1)])
        plsc.subcore_barrier()
        descs = [pltpu.async_copy(part_sh.at[pl.ds(r * C1 + s * SUB1, SUB1)],
                                  rb16.at[pl.ds(r * SUB1, SUB1)], dsem)
                 for r in range(NS)]
        for dsc in descs:
            dsc.wait()

        @plsc.parallel_loop(0, SUB1 // L, unroll=2)
        def colsum(cc):
            v = rb16[pl.ds(cc * L, L)]
            for r in range(1, NS):
                v = v + rb16[pl.ds(r * SUB1 + cc * L, L)]
            sbuf[pl.ds(cc * L, L)] = v

        pltpu.sync_copy(sbuf.at[pl.ds(0, SUB1)],
                        sc_out.at[pl.ds(base + s * SUB1, SUB1)])
        plsc.subcore_barrier()
        return 0

    lax.fori_loop(0, NCH1, chunk_body, 0)

    b2vec = b2b[...]
    base2 = wid * EP

    def mod_chunk(k, mvec):
        sb = base2 + k * C2
        pltpu.sync_copy(sc_out.at[pl.ds(sb, C2)], sbuf)
        pltpu.sync_copy(ea_hbm.at[pl.ds(sb, C2)], eabuf)

        @plsc.parallel_loop(0, C2 // L, unroll=4, carry=mvec)
        def grp(g, mv):
            sc = eabuf[pl.ds(g * L, L)] * (sbuf[pl.ds(g * L, L)] + b2vec)
            sbuf[pl.ds(g * L, L)] = sc
            return jnp.maximum(mv, sc)

        pltpu.sync_copy(sbuf, sc_out.at[pl.ds(sb, C2)])
        return grp

    mvec = lax.fori_loop(0, NCH2, mod_chunk, jnp.full((L,), NEG, jnp.float32))
    mvec = jnp.full((L,), jnp.max(mvec), jnp.float32)

    def exp_chunk(k, zvec):
        sb = base2 + k * C2
        pltpu.sync_copy(sc_out.at[pl.ds(sb, C2)], sbuf)

        @plsc.parallel_loop(0, C2 // L, unroll=4, carry=zvec)
        def grp(g, zv):
            ex = jnp.exp(sbuf[pl.ds(g * L, L)] - mvec)
            sbuf[pl.ds(g * L, L)] = ex
            return zv + ex

        pltpu.sync_copy(sbuf, sc_out.at[pl.ds(sb, C2)])
        return grp

    zvec = lax.fori_loop(0, NCH2, exp_chunk, jnp.zeros((L,), jnp.float32))
    tmp16[...] = mvec
    pltpu.sync_copy(tmp16, ms_out.at[pl.ds(wid * L, L)])
    tmp16[...] = jnp.full((L,), jnp.sum(zvec), jnp.float32)
    pltpu.sync_copy(tmp16, ms_out.at[pl.ds(NW * L + wid * L, L)])


def _sc_b_body(sc_hbm, ms_hbm, dst_hbm,
               w_out, t2_out,
               msb, sbuf, dst2b, tloc, rb16, dsem,
               part_sh):
    c = lax.axis_index("c")
    s = lax.axis_index("s")
    wid = c * NS + s
    base2 = wid * EP

    pltpu.sync_copy(ms_hbm, msb)
    mv = msb[pl.ds(0, L)]
    for r in range(1, NW):
        mv = jnp.maximum(mv, msb[pl.ds(r * L, L)])
    zv = jnp.zeros((L,), jnp.float32)
    for r in range(NW):
        zv = zv + msb[pl.ds(NW * L + r * L, L)] * jnp.exp(msb[pl.ds(r * L, L)] - mv)
    myscale = jnp.exp(msb[pl.ds(wid * L, L)] - mv) / zv

    @plsc.parallel_loop(0, NPAD // L, unroll=4)
    def zt(i):
        tloc[pl.ds(i * L, L)] = jnp.zeros((L,), jnp.float32)

    def norm_chunk(k, _):
        sb = base2 + k * C2
        pltpu.sync_copy(sc_hbm.at[pl.ds(sb, C2)], sbuf)
        pltpu.sync_copy(dst_hbm.at[pl.ds(sb, C2)], dst2b)

        @plsc.parallel_loop(0, C2 // L, unroll=4)
        def grp(g):
            wv = sbuf[pl.ds(g * L, L)] * myscale
            sbuf[pl.ds(g * L, L)] = wv
            dv = dst2b[pl.ds(g * L, L)]
            plsc.addupdate_scatter(tloc, [dv], wv)

        pltpu.sync_copy(sbuf, w_out.at[pl.ds(sb, C2)])
        return 0

    lax.fori_loop(0, NCH2, norm_chunk, 0)

    for r4 in range(NPAD // TC1):
        pltpu.sync_copy(tloc.at[pl.ds(r4 * TC1, TC1)],
                        part_sh.at[pl.ds(s * TC1, TC1)])
        plsc.subcore_barrier()
        descs = [pltpu.async_copy(part_sh.at[pl.ds(r * TC1 + s * TSUB, TSUB)],
                                  rb16.at[pl.ds(r * TSUB, TSUB)], dsem)
                 for r in range(NS)]
        for dsc in descs:
            dsc.wait()

        @plsc.parallel_loop(0, TSUB // L, unroll=2)
        def tsum(cc):
            v = rb16[pl.ds(cc * L, L)]
            for r in range(1, NS):
                v = v + rb16[pl.ds(r * TSUB + cc * L, L)]
            sbuf[pl.ds(cc * L, L)] = v

        pltpu.sync_copy(sbuf.at[pl.ds(0, TSUB)],
                        t2_out.at[pl.ds(c * NPAD + r4 * TC1 + s * TSUB, TSUB)])
        plsc.subcore_barrier()


@jax.jit
def kernel(node_feats, edge_index, edge_attr, W1, b1, W2, b2):
    x = node_feats[0]
    src = edge_index[0]
    dst = edge_index[1]
    ea = edge_attr[:, 0]
    w2f = W2[:, 0]
    b1r = b1.reshape(1, H)
    b2v = jnp.broadcast_to(b2.astype(jnp.float32), (L,))

    at, bt = pl.pallas_call(
        _proj_body,
        out_shape=(
            jax.ShapeDtypeStruct((H, N), jnp.float32),
            jax.ShapeDtypeStruct((H, N), jnp.float32),
        ),
    )(x, W1, b1r)

    mesh = plsc.VectorSubcoreMesh(core_axis_name="c", subcore_axis_name="s",
                                  num_cores=NC, num_subcores=NS)
    sc_exp, ms = pl.kernel(
        _sc_a_body,
        out_type=(
            jax.ShapeDtypeStruct((E,), jnp.float32),
            jax.ShapeDtypeStruct((2 * NW * L,), jnp.float32),
        ),
        mesh=mesh,
        compiler_params=pltpu.CompilerParams(needs_layout_passes=False),
        scratch_types=[
            pltpu.VMEM((FPT * N,), jnp.float32),
            pltpu.VMEM((FPT * N,), jnp.float32),
            pltpu.VMEM((C1,), jnp.int32),
            pltpu.VMEM((C1,), jnp.int32),
            pltpu.VMEM((C1,), jnp.float32),
            pltpu.VMEM((C1,), jnp.float32),
            pltpu.VMEM((C2,), jnp.float32),
            pltpu.VMEM((C2,), jnp.float32),
            pltpu.VMEM((H,), jnp.float32),
            pltpu.VMEM((L,), jnp.float32),
            pltpu.VMEM((L,), jnp.float32),
            pltpu.SemaphoreType.DMA,
            pltpu.VMEM_SHARED((NS * C1,), jnp.float32),
        ],
    )(at.reshape(H * N), bt.reshape(H * N), src, dst, ea, w2f, b2v)

    w, t2 = pl.kernel(
        _sc_b_body,
        out_type=(
            jax.ShapeDtypeStruct((E,), jnp.float32),
            jax.ShapeDtypeStruct((NC * NPAD,), jnp.float32),
        ),
        mesh=mesh,
        compiler_params=pltpu.CompilerParams(needs_layout_passes=False),
        scratch_types=[
            pltpu.VMEM((2 * NW * L,), jnp.float32),
            pltpu.VMEM((C2,), jnp.float32),
            pltpu.VMEM((C2,), jnp.int32),
            pltpu.VMEM((NPAD,), jnp.float32),
            pltpu.VMEM((TC1,), jnp.float32),
            pltpu.SemaphoreType.DMA,
            pltpu.VMEM_SHARED((NS * TC1,), jnp.float32),
        ],
    )(sc_exp, ms, dst)

    gf = pl.pallas_call(
        _agg_body,
        out_shape=jax.ShapeDtypeStruct((1, D), jnp.float32),
    )(t2.reshape(NC, NPAD), x)

    return gf, w.reshape(1, E)
